# Initial kernel scaffold; baseline (speedup 1.0000x reference)
import jax
import jax.numpy as jnp
from jax.experimental import pallas as pl


def kernel(h, edge_index, W, b):
    raise NotImplementedError("write your pallas kernel here")


# speedup vs baseline: 4.7993x; 4.7993x over previous
import functools

import jax
import jax.numpy as jnp
from jax import lax
from jax.experimental import pallas as pl
from jax.experimental.pallas import tpu as pltpu
from jax.experimental.pallas import tpu_sc as plsc

N_NODES = 10000
N_EDGES = 320000
D_IN = 128
D_OUT = 128

NC = 2
NS = 16
NW = NC * NS
CH = 128
DEXT = D_IN + 16
NACC = 10240
ROWS_PER_TILE = NACC // NS
EPT = -(-N_EDGES // (NW * CH)) * CH
CPT = EPT // CH
EPAD = EPT * NW
DUMMY_DST = NACC - 1


def _sc_body(hext_hbm, src_hbm, dst_hbm, zeros_hbm, acc_out,
             src_v, dst_v, rows_v, acc_sh, gsem, ssem):
    c = lax.axis_index("c")
    s = lax.axis_index("s")
    wid = s * NC + c
    my_rows = pl.ds(s * ROWS_PER_TILE, ROWS_PER_TILE)
    pltpu.sync_copy(zeros_hbm, acc_sh.at[my_rows])
    pltpu.sync_copy(src_hbm.at[wid], src_v)
    pltpu.sync_copy(dst_hbm.at[wid], dst_v)
    plsc.subcore_barrier()

    def chunk(j, carry):
        pltpu.async_copy(hext_hbm.at[src_v.at[j]], rows_v, gsem).wait()
        pltpu.async_copy(rows_v, acc_sh.at[dst_v.at[j]], ssem, add=True).wait()
        return carry

    lax.fori_loop(0, CPT, chunk, 0)
    plsc.subcore_barrier()
    pltpu.sync_copy(acc_sh.at[my_rows], acc_out.at[c, my_rows])


def _sc_aggregate(hext, src3, dst3, zeros_slab):
    mesh = plsc.VectorSubcoreMesh(core_axis_name="c", subcore_axis_name="s")
    f = pl.kernel(
        _sc_body,
        out_type=jax.ShapeDtypeStruct((NC, NACC, DEXT), jnp.float32),
        mesh=mesh,
        compiler_params=pltpu.CompilerParams(use_tc_tiling_on_sc=False),
        scratch_types=[
            pltpu.VMEM((CPT, CH), jnp.int32),
            pltpu.VMEM((CPT, CH), jnp.int32),
            pltpu.VMEM((CH, DEXT), jnp.float32),
            pltpu.VMEM_SHARED((NACC, DEXT), jnp.float32),
            pltpu.SemaphoreType.DMA,
            pltpu.SemaphoreType.DMA,
        ],
    )
    return f(hext, src3, dst3, zeros_slab)


def _tc_body(h_ref, acc_ref, w_ref, b_ref, o_ref):
    a0 = acc_ref[0]
    a1 = acc_ref[1]
    sums = a0[:, :D_IN] + a1[:, :D_IN]
    cnt = a0[:, D_IN:D_IN + 1] + a1[:, D_IN:D_IN + 1]
    neigh = sums / jnp.maximum(cnt, 1.0)
    r = lax.dot_general(h_ref[...], w_ref[:, :D_IN],
                        (((1,), (1,)), ((), ())),
                        preferred_element_type=jnp.float32)
    r = r + lax.dot_general(neigh, w_ref[:, D_IN:],
                            (((1,), (1,)), ((), ())),
                            preferred_element_type=jnp.float32)
    o_ref[...] = jnp.maximum(r + b_ref[...], 0.0)


def _tc_finish(h_pad, acc, W, b2, interpret=False):
    R = 1024
    grid = (NACC // R,)
    return pl.pallas_call(
        _tc_body,
        grid=grid,
        in_specs=[
            pl.BlockSpec((R, D_IN), lambda i: (i, 0)),
            pl.BlockSpec((NC, R, DEXT), lambda i: (0, i, 0)),
            pl.BlockSpec((D_IN, 2 * D_IN), lambda i: (0, 0)),
            pl.BlockSpec((1, D_OUT), lambda i: (0, 0)),
        ],
        out_specs=pl.BlockSpec((R, D_OUT), lambda i: (i, 0)),
        out_shape=jax.ShapeDtypeStruct((NACC, D_OUT), jnp.float32),
        interpret=interpret,
    )(h_pad, acc, W, b2)


def kernel(h, edge_index, W, b):
    src = edge_index[0].astype(jnp.int32)
    dst = edge_index[1].astype(jnp.int32)
    pad = EPAD - N_EDGES
    src3 = jnp.concatenate([src, jnp.zeros((pad,), jnp.int32)]).reshape(NW, CPT, CH)
    dst3 = jnp.concatenate([dst, jnp.full((pad,), DUMMY_DST, jnp.int32)]).reshape(NW, CPT, CH)
    ones_col = jnp.concatenate(
        [jnp.ones((N_NODES, 1), jnp.float32),
         jnp.zeros((N_NODES, DEXT - D_IN - 1), jnp.float32)], axis=1)
    hext = jnp.concatenate([h, ones_col], axis=1)
    zeros_slab = jnp.zeros((ROWS_PER_TILE, DEXT), jnp.float32)

    acc = _sc_aggregate(hext, src3, dst3, zeros_slab)

    h_pad = jnp.concatenate([h, jnp.zeros((NACC - N_NODES, D_IN), jnp.float32)])
    out = _tc_finish(h_pad, acc, W, b.reshape(1, D_OUT))
    return out[:N_NODES]


# speedup vs baseline: 5.4669x; 1.1391x over previous
import functools

import jax
import jax.numpy as jnp
from jax import lax
from jax.experimental import pallas as pl
from jax.experimental.pallas import tpu as pltpu
from jax.experimental.pallas import tpu_sc as plsc

N_NODES = 10000
N_EDGES = 320000
D_IN = 128
D_OUT = 128

NC = 2
NS = 16
NW = NC * NS
CH = 64
DEXT = D_IN + 16
NACC = 10112
ROWS_PER_TILE = NACC // NS
CPT = 2 * (-(-N_EDGES // (NW * CH * 2)))
EPT = CPT * CH
EPAD = EPT * NW
DUMMY_DST = NACC - 1


def _sc_body(hext_hbm, src_hbm, dst_hbm, zeros_hbm, acc_out,
             src_v, dst_v, rows_a, rows_b, acc_sh, gsem, ssem):
    c = lax.axis_index("c")
    s = lax.axis_index("s")
    wid = s * NC + c
    my_rows = pl.ds(s * ROWS_PER_TILE, ROWS_PER_TILE)
    pltpu.sync_copy(zeros_hbm, acc_sh.at[my_rows])
    pltpu.sync_copy(src_hbm.at[wid], src_v)
    pltpu.sync_copy(dst_hbm.at[wid], dst_v)
    plsc.subcore_barrier()

    def fire_gather(j, buf):
        pltpu.async_copy(hext_hbm.at[src_v.at[j]], buf, gsem)

    def wait_gather(j, buf):
        pltpu.make_async_copy(hext_hbm.at[src_v.at[j]], buf, gsem).wait()

    def fire_scatter(j, buf):
        pltpu.async_copy(buf, acc_sh.at[dst_v.at[j]], ssem, add=True)

    def wait_scatter(j, buf):
        pltpu.make_async_copy(buf, acc_sh.at[dst_v.at[j]], ssem).wait()

    def step(i, carry):
        j0 = 2 * i
        @pl.when(i >= 1)
        def _():
            wait_scatter(j0 - 1, rows_b)
        fire_gather(j0 + 1, rows_b)
        wait_gather(j0, rows_a)
        fire_scatter(j0, rows_a)

        wait_scatter(j0, rows_a)
        @pl.when(i < CPT // 2 - 1)
        def _():
            fire_gather(j0 + 2, rows_a)
        wait_gather(j0 + 1, rows_b)
        fire_scatter(j0 + 1, rows_b)
        return carry

    fire_gather(0, rows_a)
    lax.fori_loop(0, CPT // 2, step, 0)
    wait_scatter(CPT - 1, rows_b)
    plsc.subcore_barrier()
    pltpu.sync_copy(acc_sh.at[my_rows], acc_out.at[c, my_rows])


def _sc_aggregate(hext, src3, dst3, zeros_slab):
    mesh = plsc.VectorSubcoreMesh(core_axis_name="c", subcore_axis_name="s")
    f = pl.kernel(
        _sc_body,
        out_type=jax.ShapeDtypeStruct((NC, NACC, DEXT), jnp.float32),
        mesh=mesh,
        compiler_params=pltpu.CompilerParams(use_tc_tiling_on_sc=False),
        scratch_types=[
            pltpu.VMEM((CPT, CH), jnp.int32),
            pltpu.VMEM((CPT, CH), jnp.int32),
            pltpu.VMEM((CH, DEXT), jnp.float32),
            pltpu.VMEM((CH, DEXT), jnp.float32),
            pltpu.VMEM_SHARED((NACC, DEXT), jnp.float32),
            pltpu.SemaphoreType.DMA,
            pltpu.SemaphoreType.DMA,
        ],
    )
    return f(hext, src3, dst3, zeros_slab)


def _tc_body(h_ref, acc_ref, w_ref, b_ref, o_ref):
    a0 = acc_ref[0]
    a1 = acc_ref[1]
    sums = a0[:, :D_IN] + a1[:, :D_IN]
    cnt = a0[:, D_IN:D_IN + 1] + a1[:, D_IN:D_IN + 1]
    neigh = sums / jnp.maximum(cnt, 1.0)
    r = lax.dot_general(h_ref[...], w_ref[:, :D_IN],
                        (((1,), (1,)), ((), ())),
                        preferred_element_type=jnp.float32)
    r = r + lax.dot_general(neigh, w_ref[:, D_IN:],
                            (((1,), (1,)), ((), ())),
                            preferred_element_type=jnp.float32)
    o_ref[...] = jnp.maximum(r + b_ref[...], 0.0)


def _tc_finish(h_pad, acc, W, b2, interpret=False):
    R = 632
    grid = (NACC // R,)
    return pl.pallas_call(
        _tc_body,
        grid=grid,
        in_specs=[
            pl.BlockSpec((R, D_IN), lambda i: (i, 0)),
            pl.BlockSpec((NC, R, DEXT), lambda i: (0, i, 0)),
            pl.BlockSpec((D_IN, 2 * D_IN), lambda i: (0, 0)),
            pl.BlockSpec((1, D_OUT), lambda i: (0, 0)),
        ],
        out_specs=pl.BlockSpec((R, D_OUT), lambda i: (i, 0)),
        out_shape=jax.ShapeDtypeStruct((NACC, D_OUT), jnp.float32),
        interpret=interpret,
    )(h_pad, acc, W, b2)


def kernel(h, edge_index, W, b):
    src = edge_index[0].astype(jnp.int32)
    dst = edge_index[1].astype(jnp.int32)
    pad = EPAD - N_EDGES
    src3 = jnp.concatenate([src, jnp.zeros((pad,), jnp.int32)]).reshape(NW, CPT, CH)
    dst_pad = N_NODES + (jnp.arange(pad, dtype=jnp.int32) % (NACC - N_NODES))
    dst3 = jnp.concatenate([dst, dst_pad]).reshape(NW, CPT, CH)
    ones_col = jnp.concatenate(
        [jnp.ones((N_NODES, 1), jnp.float32),
         jnp.zeros((N_NODES, DEXT - D_IN - 1), jnp.float32)], axis=1)
    hext = jnp.concatenate([h, ones_col], axis=1)
    zeros_slab = jnp.zeros((ROWS_PER_TILE, DEXT), jnp.float32)

    acc = _sc_aggregate(hext, src3, dst3, zeros_slab)

    h_pad = jnp.concatenate([h, jnp.zeros((NACC - N_NODES, D_IN), jnp.float32)])
    out = _tc_finish(h_pad, acc, W, b.reshape(1, D_OUT))
    return out[:N_NODES]
